# baseline (speedup 1.0000x reference)
import functools

import jax
import jax.numpy as jnp
from jax import lax
from jax.experimental import pallas as pl
from jax.experimental.pallas import tpu as pltpu
from jax.experimental.pallas import tpu_sc as plsc

EMB = 64
BATCH = 16384
NC = 2
NS = 16
NW = NC * NS
BPW = BATCH // NW
CHUNK = 128
NCH = BPW // CHUNK

_mesh = plsc.VectorSubcoreMesh(
    core_axis_name="c", subcore_axis_name="s", num_cores=NC, num_subcores=NS
)

_out = jax.ShapeDtypeStruct((NW, BPW, EMB), jnp.float32)


@functools.partial(
    pl.kernel,
    out_type=[_out, _out, _out],
    mesh=_mesh,
    compiler_params=pltpu.CompilerParams(use_tc_tiling_on_sc=False),
    scratch_types=[
        pltpu.VMEM((NCH, CHUNK), jnp.int32),
        pltpu.VMEM((NCH, CHUNK), jnp.int32),
        pltpu.VMEM((NCH, CHUNK), jnp.int32),
        pltpu.VMEM((BPW, EMB), jnp.float32),
        pltpu.VMEM((BPW, EMB), jnp.float32),
        pltpu.VMEM((BPW, EMB), jnp.float32),
        pltpu.SemaphoreType.DMA,
    ],
)
def _gather3(users_hbm, pos_hbm, neg_hbm, utab_hbm, itab_hbm,
             out_u, out_p, out_n,
             idx_u, idx_p, idx_n, rows_u, rows_p, rows_n, sem):
    wid = lax.axis_index("s") * NC + lax.axis_index("c")

    pltpu.sync_copy(users_hbm.at[wid], idx_u)
    pltpu.sync_copy(pos_hbm.at[wid], idx_p)
    pltpu.sync_copy(neg_hbm.at[wid], idx_n)

    copies = []
    for tab, idx, rows in ((utab_hbm, idx_u, rows_u),
                           (itab_hbm, idx_p, rows_p),
                           (itab_hbm, idx_n, rows_n)):
        for j in range(NCH):
            copies.append(
                pltpu.async_copy(
                    tab.at[idx.at[j]], rows.at[pl.ds(j * CHUNK, CHUNK)], sem
                )
            )
    for c in copies:
        c.wait()

    pltpu.sync_copy(rows_u, out_u.at[wid])
    pltpu.sync_copy(rows_p, out_p.at[wid])
    pltpu.sync_copy(rows_n, out_n.at[wid])


def kernel(users, pos_items, neg_items, user_table, item_table):
    u = users.astype(jnp.int32).reshape(NW, NCH, CHUNK)
    p = pos_items.astype(jnp.int32).reshape(NW, NCH, CHUNK)
    n = neg_items.astype(jnp.int32).reshape(NW, NCH, CHUNK)
    out_u, out_p, out_n = _gather3(u, p, n, user_table, item_table)
    return (out_u.reshape(BATCH, EMB),
            out_p.reshape(BATCH, EMB),
            out_n.reshape(BATCH, EMB))
